# initial kernel scaffold (unmeasured)
import functools

import jax
import jax.numpy as jnp
from jax import lax
from jax.experimental import pallas as pl
from jax.experimental.pallas import tpu as pltpu

N_DEV = 32
H = 32
DH = 128
DR = 64


def _ring_reduce_scatter_kv(kv):
    n, two, T, dh = kv.shape

    def body(kv_ref, out_ref, comm_ref, send_sems, recv_sems, credit_sem):
        my = lax.axis_index("i")
        left = lax.rem(my + N_DEV - 1, N_DEV)
        right = lax.rem(my + 1, N_DEV)

        barrier_sem = pltpu.get_barrier_semaphore()
        for nbr in (left, right):
            pl.semaphore_signal(
                barrier_sem, inc=1, device_id=(nbr,),
                device_id_type=pl.DeviceIdType.MESH,
            )
        pl.semaphore_wait(barrier_sem, 2)

        comm_ref[0] = kv_ref[left]

        for s in range(N_DEV - 1):
            send_slot = s % 2
            recv_slot = (s + 1) % 2
            if s >= 1:
                pl.semaphore_wait(credit_sem, 1)
            rdma = pltpu.make_async_remote_copy(
                src_ref=comm_ref.at[send_slot],
                dst_ref=comm_ref.at[recv_slot],
                send_sem=send_sems.at[send_slot],
                recv_sem=recv_sems.at[recv_slot],
                device_id=(right,),
                device_id_type=pl.DeviceIdType.MESH,
            )
            rdma.start()
            rdma.wait()
            j = lax.rem(my + 2 * N_DEV - 2 - s, N_DEV)
            if s < N_DEV - 2:
                comm_ref[recv_slot] = comm_ref[recv_slot] + kv_ref[j]
            else:
                out_ref[...] = comm_ref[recv_slot] + kv_ref[j]
            pl.semaphore_signal(
                credit_sem, inc=1, device_id=(left,),
                device_id_type=pl.DeviceIdType.MESH,
            )

    return pl.pallas_call(
        body,
        out_shape=jax.ShapeDtypeStruct((two, T, dh), kv.dtype),
        in_specs=[pl.BlockSpec(memory_space=pltpu.VMEM)],
        out_specs=pl.BlockSpec(memory_space=pltpu.VMEM),
        scratch_shapes=[
            pltpu.VMEM((2, two, T, dh), kv.dtype),
            pltpu.SemaphoreType.DMA((2,)),
            pltpu.SemaphoreType.DMA((2,)),
            pltpu.SemaphoreType.REGULAR,
        ],
        compiler_params=pltpu.CompilerParams(collective_id=0),
    )(kv)


def _ring_all_gather_o(o):
    T, dh = o.shape

    def body(x_ref, out_ref, comm_ref, send_sems, recv_sems, credit_sem):
        my = lax.axis_index("i")
        left = lax.rem(my + N_DEV - 1, N_DEV)
        right = lax.rem(my + 1, N_DEV)

        barrier_sem = pltpu.get_barrier_semaphore()
        for nbr in (left, right):
            pl.semaphore_signal(
                barrier_sem, inc=1, device_id=(nbr,),
                device_id_type=pl.DeviceIdType.MESH,
            )
        pl.semaphore_wait(barrier_sem, 2)

        out_ref[my] = x_ref[...]
        comm_ref[0] = x_ref[...]

        for h in range(N_DEV - 1):
            send_slot = h % 2
            recv_slot = (h + 1) % 2
            if h >= 1:
                pl.semaphore_wait(credit_sem, 1)
            rdma = pltpu.make_async_remote_copy(
                src_ref=comm_ref.at[send_slot],
                dst_ref=comm_ref.at[recv_slot],
                send_sem=send_sems.at[send_slot],
                recv_sem=recv_sems.at[recv_slot],
                device_id=(right,),
                device_id_type=pl.DeviceIdType.MESH,
            )
            rdma.start()
            rdma.wait()
            origin = lax.rem(my + 2 * N_DEV - 1 - h, N_DEV)
            out_ref[origin] = comm_ref[recv_slot]
            pl.semaphore_signal(
                credit_sem, inc=1, device_id=(left,),
                device_id_type=pl.DeviceIdType.MESH,
            )

    return pl.pallas_call(
        body,
        out_shape=jax.ShapeDtypeStruct((N_DEV, T, dh), o.dtype),
        in_specs=[pl.BlockSpec(memory_space=pltpu.VMEM)],
        out_specs=pl.BlockSpec(memory_space=pltpu.VMEM),
        scratch_shapes=[
            pltpu.VMEM((2, T, dh), o.dtype),
            pltpu.SemaphoreType.DMA((2,)),
            pltpu.SemaphoreType.DMA((2,)),
            pltpu.SemaphoreType.REGULAR,
        ],
        compiler_params=pltpu.CompilerParams(collective_id=1),
    )(o)


def kernel(x, Wdkv, Wuk, Wuv, Wq, Wqr, Wkr, Wo):
    B, S, D = x.shape
    T = B * S
    xt = x.reshape(T, D)

    c = xt @ Wdkv
    kp = c @ Wuk
    vp = c @ Wuv
    kv = jnp.stack([kp, vp])
    kv = kv.reshape(2, T, N_DEV, DH).transpose(2, 0, 1, 3)

    kv_head = _ring_reduce_scatter_kv(kv)
    K = kv_head[0].reshape(B, S, DH)
    V = kv_head[1].reshape(B, S, DH)

    my = lax.axis_index("i")
    Wq_h = lax.dynamic_slice_in_dim(Wq, my * DH, DH, axis=1)
    Wqr_h = lax.dynamic_slice_in_dim(Wqr, my * DR, DR, axis=1)

    Q = (xt @ Wq_h).reshape(B, S, DH)
    Qr = (xt @ Wqr_h).reshape(B, S, DR)
    Kr = (xt @ Wkr).reshape(B, S, DR)

    scale = (DH + DR) ** -0.5
    scores = (
        jnp.einsum("bsd,btd->bst", Q, K)
        + jnp.einsum("bsr,btr->bst", Qr, Kr)
    ) * scale
    m = scores.max(-1, keepdims=True)
    P = jnp.exp(scores - m)
    P = P / P.sum(-1, keepdims=True)
    O = jnp.einsum("bst,btd->bsd", P, V)

    og = _ring_all_gather_o(O.reshape(T, DH))
    O_full = og.transpose(1, 0, 2).reshape(T, H * DH)
    out = O_full @ Wo
    return out.reshape(B, S, D)


# baseline (device time: 1123454 ns/iter reference)
import functools

import jax
import jax.numpy as jnp
from jax import lax
from jax.experimental import pallas as pl
from jax.experimental.pallas import tpu as pltpu

N_DEV = 32
H = 32
DH = 128
DR = 64


def _ring_reduce_scatter_kv(kv):
    n, two, T, dh = kv.shape

    def body(kv_ref, out_ref, comm_ref, send_sems, recv_sems, credit_sem):
        my = lax.axis_index("i")
        left = lax.rem(my + N_DEV - 1, N_DEV)
        right = lax.rem(my + 1, N_DEV)

        barrier_sem = pltpu.get_barrier_semaphore()
        for nbr in (left, right):
            pl.semaphore_signal(
                barrier_sem, inc=1, device_id=(nbr,),
                device_id_type=pl.DeviceIdType.MESH,
            )
        pl.semaphore_wait(barrier_sem, 2)

        comm_ref[0] = kv_ref[left]

        for s in range(N_DEV - 1):
            send_slot = s % 2
            recv_slot = (s + 1) % 2
            if s >= 1:
                pl.semaphore_wait(credit_sem, 1)
            rdma = pltpu.make_async_remote_copy(
                src_ref=comm_ref.at[send_slot],
                dst_ref=comm_ref.at[recv_slot],
                send_sem=send_sems.at[send_slot],
                recv_sem=recv_sems.at[recv_slot],
                device_id=(right,),
                device_id_type=pl.DeviceIdType.MESH,
            )
            rdma.start()
            rdma.wait()
            j = lax.rem(my + 2 * N_DEV - 2 - s, N_DEV)
            if s < N_DEV - 2:
                comm_ref[recv_slot] = comm_ref[recv_slot] + kv_ref[j]
            else:
                out_ref[...] = comm_ref[recv_slot] + kv_ref[j]
            if s < N_DEV - 2:
                pl.semaphore_signal(
                    credit_sem, inc=1, device_id=(left,),
                    device_id_type=pl.DeviceIdType.MESH,
                )

    return pl.pallas_call(
        body,
        out_shape=jax.ShapeDtypeStruct((two, T, dh), kv.dtype),
        in_specs=[pl.BlockSpec(memory_space=pltpu.VMEM)],
        out_specs=pl.BlockSpec(memory_space=pltpu.VMEM),
        scratch_shapes=[
            pltpu.VMEM((2, two, T, dh), kv.dtype),
            pltpu.SemaphoreType.DMA((2,)),
            pltpu.SemaphoreType.DMA((2,)),
            pltpu.SemaphoreType.REGULAR,
        ],
        compiler_params=pltpu.CompilerParams(
            collective_id=0, vmem_limit_bytes=64 * 1024 * 1024
        ),
    )(kv)


def _ring_all_gather_o(o):
    T, dh = o.shape

    def body(x_ref, out_ref, comm_ref, send_sems, recv_sems, credit_sem):
        my = lax.axis_index("i")
        left = lax.rem(my + N_DEV - 1, N_DEV)
        right = lax.rem(my + 1, N_DEV)

        barrier_sem = pltpu.get_barrier_semaphore()
        for nbr in (left, right):
            pl.semaphore_signal(
                barrier_sem, inc=1, device_id=(nbr,),
                device_id_type=pl.DeviceIdType.MESH,
            )
        pl.semaphore_wait(barrier_sem, 2)

        out_ref[my] = x_ref[...]
        comm_ref[0] = x_ref[...]

        for h in range(N_DEV - 1):
            send_slot = h % 2
            recv_slot = (h + 1) % 2
            if h >= 1:
                pl.semaphore_wait(credit_sem, 1)
            rdma = pltpu.make_async_remote_copy(
                src_ref=comm_ref.at[send_slot],
                dst_ref=comm_ref.at[recv_slot],
                send_sem=send_sems.at[send_slot],
                recv_sem=recv_sems.at[recv_slot],
                device_id=(right,),
                device_id_type=pl.DeviceIdType.MESH,
            )
            rdma.start()
            rdma.wait()
            origin = lax.rem(my + 2 * N_DEV - 1 - h, N_DEV)
            out_ref[origin] = comm_ref[recv_slot]
            if h < N_DEV - 2:
                pl.semaphore_signal(
                    credit_sem, inc=1, device_id=(left,),
                    device_id_type=pl.DeviceIdType.MESH,
                )

    return pl.pallas_call(
        body,
        out_shape=jax.ShapeDtypeStruct((N_DEV, T, dh), o.dtype),
        in_specs=[pl.BlockSpec(memory_space=pltpu.VMEM)],
        out_specs=pl.BlockSpec(memory_space=pltpu.VMEM),
        scratch_shapes=[
            pltpu.VMEM((2, T, dh), o.dtype),
            pltpu.SemaphoreType.DMA((2,)),
            pltpu.SemaphoreType.DMA((2,)),
            pltpu.SemaphoreType.REGULAR,
        ],
        compiler_params=pltpu.CompilerParams(collective_id=1),
    )(o)


def kernel(x, Wdkv, Wuk, Wuv, Wq, Wqr, Wkr, Wo):
    B, S, D = x.shape
    T = B * S
    xt = x.reshape(T, D)

    c = xt @ Wdkv
    kp = c @ Wuk
    vp = c @ Wuv
    kv = jnp.stack([kp, vp])
    kv = kv.reshape(2, T, N_DEV, DH).transpose(2, 0, 1, 3)

    kv_head = _ring_reduce_scatter_kv(kv)
    K = kv_head[0].reshape(B, S, DH)
    V = kv_head[1].reshape(B, S, DH)

    my = lax.axis_index("i")
    Wq_h = lax.dynamic_slice_in_dim(Wq, my * DH, DH, axis=1)
    Wqr_h = lax.dynamic_slice_in_dim(Wqr, my * DR, DR, axis=1)

    Q = (xt @ Wq_h).reshape(B, S, DH)
    Qr = (xt @ Wqr_h).reshape(B, S, DR)
    Kr = (xt @ Wkr).reshape(B, S, DR)

    scale = (DH + DR) ** -0.5
    scores = (
        jnp.einsum("bsd,btd->bst", Q, K)
        + jnp.einsum("bsr,btr->bst", Qr, Kr)
    ) * scale
    m = scores.max(-1, keepdims=True)
    P = jnp.exp(scores - m)
    P = P / P.sum(-1, keepdims=True)
    O = jnp.einsum("bst,btd->bsd", P, V)

    og = _ring_all_gather_o(O.reshape(T, DH))
    O_full = og.transpose(1, 0, 2).reshape(T, H * DH)
    out = O_full @ Wo
    return out.reshape(B, S, D)


# device time: 968162 ns/iter; 1.1604x vs baseline; 1.1604x over previous
import functools

import jax
import jax.numpy as jnp
from jax import lax
from jax.experimental import pallas as pl
from jax.experimental.pallas import tpu as pltpu

N_DEV = 32
H = 32
DH = 128
DR = 64


def _ring_reduce_scatter_kv(kv):
    n, two, T, dh = kv.shape

    def body(kv_ref, out_ref, commk, commv, ksend, krecv, vsend, vrecv,
             creditk, creditv):
        my = lax.axis_index("i")
        left = lax.rem(my + N_DEV - 1, N_DEV)
        right = lax.rem(my + 1, N_DEV)

        barrier_sem = pltpu.get_barrier_semaphore()
        for nbr in (left, right):
            pl.semaphore_signal(
                barrier_sem, inc=1, device_id=(nbr,),
                device_id_type=pl.DeviceIdType.MESH,
            )
        pl.semaphore_wait(barrier_sem, 2)

        commk[0] = kv_ref[left, 0]
        commv[0] = kv_ref[right, 1]

        for s in range(N_DEV - 1):
            send_slot = s % 2
            recv_slot = (s + 1) % 2
            if s >= 1:
                pl.semaphore_wait(creditk, 1)
                pl.semaphore_wait(creditv, 1)
            rk = pltpu.make_async_remote_copy(
                src_ref=commk.at[send_slot],
                dst_ref=commk.at[recv_slot],
                send_sem=ksend.at[send_slot],
                recv_sem=krecv.at[recv_slot],
                device_id=(right,),
                device_id_type=pl.DeviceIdType.MESH,
            )
            rv = pltpu.make_async_remote_copy(
                src_ref=commv.at[send_slot],
                dst_ref=commv.at[recv_slot],
                send_sem=vsend.at[send_slot],
                recv_sem=vrecv.at[recv_slot],
                device_id=(left,),
                device_id_type=pl.DeviceIdType.MESH,
            )
            rk.start()
            rv.start()
            rk.wait()
            jk = lax.rem(my + 2 * N_DEV - 2 - s, N_DEV)
            if s < N_DEV - 2:
                commk[recv_slot] = commk[recv_slot] + kv_ref[jk, 0]
            else:
                out_ref[0] = commk[recv_slot] + kv_ref[jk, 0]
            rv.wait()
            jv = lax.rem(my + 2 + s, N_DEV)
            if s < N_DEV - 2:
                commv[recv_slot] = commv[recv_slot] + kv_ref[jv, 1]
            else:
                out_ref[1] = commv[recv_slot] + kv_ref[jv, 1]
            if s < N_DEV - 2:
                pl.semaphore_signal(
                    creditk, inc=1, device_id=(left,),
                    device_id_type=pl.DeviceIdType.MESH,
                )
                pl.semaphore_signal(
                    creditv, inc=1, device_id=(right,),
                    device_id_type=pl.DeviceIdType.MESH,
                )

    return pl.pallas_call(
        body,
        out_shape=jax.ShapeDtypeStruct((two, T, dh), kv.dtype),
        in_specs=[pl.BlockSpec(memory_space=pltpu.VMEM)],
        out_specs=pl.BlockSpec(memory_space=pltpu.VMEM),
        scratch_shapes=[
            pltpu.VMEM((2, T, dh), kv.dtype),
            pltpu.VMEM((2, T, dh), kv.dtype),
            pltpu.SemaphoreType.DMA((2,)),
            pltpu.SemaphoreType.DMA((2,)),
            pltpu.SemaphoreType.DMA((2,)),
            pltpu.SemaphoreType.DMA((2,)),
            pltpu.SemaphoreType.REGULAR,
            pltpu.SemaphoreType.REGULAR,
        ],
        compiler_params=pltpu.CompilerParams(
            collective_id=0, vmem_limit_bytes=64 * 1024 * 1024
        ),
    )(kv)


def _ring_all_gather_o(o):
    T, dh = o.shape

    def body(x_ref, out_ref, comm_ref, send_sems, recv_sems, credit_sem):
        my = lax.axis_index("i")
        left = lax.rem(my + N_DEV - 1, N_DEV)
        right = lax.rem(my + 1, N_DEV)

        barrier_sem = pltpu.get_barrier_semaphore()
        for nbr in (left, right):
            pl.semaphore_signal(
                barrier_sem, inc=1, device_id=(nbr,),
                device_id_type=pl.DeviceIdType.MESH,
            )
        pl.semaphore_wait(barrier_sem, 2)

        out_ref[my] = x_ref[...]
        comm_ref[0] = x_ref[...]

        for h in range(N_DEV - 1):
            send_slot = h % 2
            recv_slot = (h + 1) % 2
            if h >= 1:
                pl.semaphore_wait(credit_sem, 1)
            rdma = pltpu.make_async_remote_copy(
                src_ref=comm_ref.at[send_slot],
                dst_ref=comm_ref.at[recv_slot],
                send_sem=send_sems.at[send_slot],
                recv_sem=recv_sems.at[recv_slot],
                device_id=(right,),
                device_id_type=pl.DeviceIdType.MESH,
            )
            rdma.start()
            rdma.wait()
            origin = lax.rem(my + 2 * N_DEV - 1 - h, N_DEV)
            out_ref[origin] = comm_ref[recv_slot]
            if h < N_DEV - 2:
                pl.semaphore_signal(
                    credit_sem, inc=1, device_id=(left,),
                    device_id_type=pl.DeviceIdType.MESH,
                )

    return pl.pallas_call(
        body,
        out_shape=jax.ShapeDtypeStruct((N_DEV, T, dh), o.dtype),
        in_specs=[pl.BlockSpec(memory_space=pltpu.VMEM)],
        out_specs=pl.BlockSpec(memory_space=pltpu.VMEM),
        scratch_shapes=[
            pltpu.VMEM((2, T, dh), o.dtype),
            pltpu.SemaphoreType.DMA((2,)),
            pltpu.SemaphoreType.DMA((2,)),
            pltpu.SemaphoreType.REGULAR,
        ],
        compiler_params=pltpu.CompilerParams(collective_id=1),
    )(o)


def kernel(x, Wdkv, Wuk, Wuv, Wq, Wqr, Wkr, Wo):
    B, S, D = x.shape
    T = B * S
    xt = x.reshape(T, D)

    c = xt @ Wdkv
    kp = c @ Wuk
    vp = c @ Wuv
    kv = jnp.stack([kp, vp])
    kv = kv.reshape(2, T, N_DEV, DH).transpose(2, 0, 1, 3)

    kv_head = _ring_reduce_scatter_kv(kv)
    K = kv_head[0].reshape(B, S, DH)
    V = kv_head[1].reshape(B, S, DH)

    my = lax.axis_index("i")
    Wq_h = lax.dynamic_slice_in_dim(Wq, my * DH, DH, axis=1)
    Wqr_h = lax.dynamic_slice_in_dim(Wqr, my * DR, DR, axis=1)

    Q = (xt @ Wq_h).reshape(B, S, DH)
    Qr = (xt @ Wqr_h).reshape(B, S, DR)
    Kr = (xt @ Wkr).reshape(B, S, DR)

    scale = (DH + DR) ** -0.5
    scores = (
        jnp.einsum("bsd,btd->bst", Q, K)
        + jnp.einsum("bsr,btr->bst", Qr, Kr)
    ) * scale
    m = scores.max(-1, keepdims=True)
    P = jnp.exp(scores - m)
    P = P / P.sum(-1, keepdims=True)
    O = jnp.einsum("bst,btd->bsd", P, V)

    og = _ring_all_gather_o(O.reshape(T, DH))
    O_full = og.transpose(1, 0, 2).reshape(T, H * DH)
    out = O_full @ Wo
    return out.reshape(B, S, D)


# device time: 788607 ns/iter; 1.4246x vs baseline; 1.2277x over previous
import functools

import jax
import jax.numpy as jnp
from jax import lax
from jax.experimental import pallas as pl
from jax.experimental.pallas import tpu as pltpu

N_DEV = 32
H = 32
DH = 128
DR = 64


def _ring_reduce_scatter_kv(kv):
    n, two, T, dh = kv.shape

    def body(kv_ref, out_ref, commk, commv, ksend, krecv, vsend, vrecv,
             creditk, creditv):
        my = lax.axis_index("i")
        left = lax.rem(my + N_DEV - 1, N_DEV)
        right = lax.rem(my + 1, N_DEV)

        barrier_sem = pltpu.get_barrier_semaphore()
        for nbr in (left, right):
            pl.semaphore_signal(
                barrier_sem, inc=1, device_id=(nbr,),
                device_id_type=pl.DeviceIdType.MESH,
            )
        pl.semaphore_wait(barrier_sem, 2)

        commk[0] = kv_ref[left, 0]
        commv[0] = kv_ref[right, 1]

        for s in range(N_DEV - 1):
            send_slot = s % 2
            recv_slot = (s + 1) % 2
            if s >= 1:
                pl.semaphore_wait(creditk, 1)
                pl.semaphore_wait(creditv, 1)
            rk = pltpu.make_async_remote_copy(
                src_ref=commk.at[send_slot],
                dst_ref=commk.at[recv_slot],
                send_sem=ksend.at[send_slot],
                recv_sem=krecv.at[recv_slot],
                device_id=(right,),
                device_id_type=pl.DeviceIdType.MESH,
            )
            rv = pltpu.make_async_remote_copy(
                src_ref=commv.at[send_slot],
                dst_ref=commv.at[recv_slot],
                send_sem=vsend.at[send_slot],
                recv_sem=vrecv.at[recv_slot],
                device_id=(left,),
                device_id_type=pl.DeviceIdType.MESH,
            )
            rk.start()
            rv.start()
            rk.wait()
            jk = lax.rem(my + 2 * N_DEV - 2 - s, N_DEV)
            if s < N_DEV - 2:
                commk[recv_slot] = commk[recv_slot] + kv_ref[jk, 0]
            else:
                out_ref[0] = commk[recv_slot] + kv_ref[jk, 0]
            rv.wait()
            jv = lax.rem(my + 2 + s, N_DEV)
            if s < N_DEV - 2:
                commv[recv_slot] = commv[recv_slot] + kv_ref[jv, 1]
            else:
                out_ref[1] = commv[recv_slot] + kv_ref[jv, 1]
            if s < N_DEV - 2:
                pl.semaphore_signal(
                    creditk, inc=1, device_id=(left,),
                    device_id_type=pl.DeviceIdType.MESH,
                )
                pl.semaphore_signal(
                    creditv, inc=1, device_id=(right,),
                    device_id_type=pl.DeviceIdType.MESH,
                )

    return pl.pallas_call(
        body,
        out_shape=jax.ShapeDtypeStruct((two, T, dh), kv.dtype),
        in_specs=[pl.BlockSpec(memory_space=pltpu.VMEM)],
        out_specs=pl.BlockSpec(memory_space=pltpu.VMEM),
        scratch_shapes=[
            pltpu.VMEM((2, T, dh), kv.dtype),
            pltpu.VMEM((2, T, dh), kv.dtype),
            pltpu.SemaphoreType.DMA((2,)),
            pltpu.SemaphoreType.DMA((2,)),
            pltpu.SemaphoreType.DMA((2,)),
            pltpu.SemaphoreType.DMA((2,)),
            pltpu.SemaphoreType.REGULAR,
            pltpu.SemaphoreType.REGULAR,
        ],
        compiler_params=pltpu.CompilerParams(
            collective_id=0, vmem_limit_bytes=64 * 1024 * 1024
        ),
    )(kv)


M_R = N_DEV // 2
M_L = N_DEV - 1 - M_R


def _ring_all_gather_o(o):
    T, dh = o.shape

    def body(x_ref, out_ref, commr, comml, rsend, rrecv, lsend, lrecv,
             creditr, creditl):
        my = lax.axis_index("i")
        left = lax.rem(my + N_DEV - 1, N_DEV)
        right = lax.rem(my + 1, N_DEV)

        barrier_sem = pltpu.get_barrier_semaphore()
        for nbr in (left, right):
            pl.semaphore_signal(
                barrier_sem, inc=1, device_id=(nbr,),
                device_id_type=pl.DeviceIdType.MESH,
            )
        pl.semaphore_wait(barrier_sem, 2)

        out_ref[my] = x_ref[...]
        commr[0] = x_ref[...]
        comml[0] = x_ref[...]

        for h in range(M_R):
            send_slot = h % 2
            recv_slot = (h + 1) % 2
            if h >= 1:
                pl.semaphore_wait(creditr, 1)
                if h <= M_L - 1:
                    pl.semaphore_wait(creditl, 1)
            rr = pltpu.make_async_remote_copy(
                src_ref=commr.at[send_slot],
                dst_ref=commr.at[recv_slot],
                send_sem=rsend.at[send_slot],
                recv_sem=rrecv.at[recv_slot],
                device_id=(right,),
                device_id_type=pl.DeviceIdType.MESH,
            )
            rr.start()
            if h <= M_L - 1:
                rl = pltpu.make_async_remote_copy(
                    src_ref=comml.at[send_slot],
                    dst_ref=comml.at[recv_slot],
                    send_sem=lsend.at[send_slot],
                    recv_sem=lrecv.at[recv_slot],
                    device_id=(left,),
                    device_id_type=pl.DeviceIdType.MESH,
                )
                rl.start()
            rr.wait()
            origin_r = lax.rem(my + N_DEV - 1 - h, N_DEV)
            out_ref[origin_r] = commr[recv_slot]
            if h <= M_L - 1:
                rl.wait()
                origin_l = lax.rem(my + 1 + h, N_DEV)
                out_ref[origin_l] = comml[recv_slot]
            if h < M_R - 1:
                pl.semaphore_signal(
                    creditr, inc=1, device_id=(left,),
                    device_id_type=pl.DeviceIdType.MESH,
                )
            if h < M_L - 1:
                pl.semaphore_signal(
                    creditl, inc=1, device_id=(right,),
                    device_id_type=pl.DeviceIdType.MESH,
                )

    return pl.pallas_call(
        body,
        out_shape=jax.ShapeDtypeStruct((N_DEV, T, dh), o.dtype),
        in_specs=[pl.BlockSpec(memory_space=pltpu.VMEM)],
        out_specs=pl.BlockSpec(memory_space=pltpu.VMEM),
        scratch_shapes=[
            pltpu.VMEM((2, T, dh), o.dtype),
            pltpu.VMEM((2, T, dh), o.dtype),
            pltpu.SemaphoreType.DMA((2,)),
            pltpu.SemaphoreType.DMA((2,)),
            pltpu.SemaphoreType.DMA((2,)),
            pltpu.SemaphoreType.DMA((2,)),
            pltpu.SemaphoreType.REGULAR,
            pltpu.SemaphoreType.REGULAR,
        ],
        compiler_params=pltpu.CompilerParams(
            collective_id=1, vmem_limit_bytes=64 * 1024 * 1024
        ),
    )(o)


def kernel(x, Wdkv, Wuk, Wuv, Wq, Wqr, Wkr, Wo):
    B, S, D = x.shape
    T = B * S
    xt = x.reshape(T, D)

    c = xt @ Wdkv
    kp = c @ Wuk
    vp = c @ Wuv
    kv = jnp.stack([kp, vp])
    kv = kv.reshape(2, T, N_DEV, DH).transpose(2, 0, 1, 3)

    kv_head = _ring_reduce_scatter_kv(kv)
    K = kv_head[0].reshape(B, S, DH)
    V = kv_head[1].reshape(B, S, DH)

    my = lax.axis_index("i")
    Wq_h = lax.dynamic_slice_in_dim(Wq, my * DH, DH, axis=1)
    Wqr_h = lax.dynamic_slice_in_dim(Wqr, my * DR, DR, axis=1)

    Q = (xt @ Wq_h).reshape(B, S, DH)
    Qr = (xt @ Wqr_h).reshape(B, S, DR)
    Kr = (xt @ Wkr).reshape(B, S, DR)

    scale = (DH + DR) ** -0.5
    scores = (
        jnp.einsum("bsd,btd->bst", Q, K)
        + jnp.einsum("bsr,btr->bst", Qr, Kr)
    ) * scale
    m = scores.max(-1, keepdims=True)
    P = jnp.exp(scores - m)
    P = P / P.sum(-1, keepdims=True)
    O = jnp.einsum("bst,btd->bsd", P, V)

    og = _ring_all_gather_o(O.reshape(T, DH))
    O_full = og.transpose(1, 0, 2).reshape(T, H * DH)
    out = O_full @ Wo
    return out.reshape(B, S, D)
